# baseline (speedup 1.0000x reference)
import functools

import jax
import jax.numpy as jnp
from jax import lax
from jax.experimental import pallas as pl
from jax.experimental.pallas import tpu as pltpu
from jax.experimental.pallas import tpu_sc as plsc

_L = 16
_D = 64
_NCHUNK = _D // _L


def _full(v, dtype=jnp.float32):
    return jnp.full((_L,), v, dtype)


def _rsqrt16(v):
    i = lax.bitcast_convert_type(v, jnp.int32)
    i = _full(0x5F3759DF, jnp.int32) - lax.shift_right_arithmetic(
        i, _full(1, jnp.int32)
    )
    y = lax.bitcast_convert_type(i, jnp.float32)
    half_v = _full(0.5) * v
    three_half = _full(1.5)
    for _ in range(4):
        y = y * (three_half - half_v * y * y)
    return y


def _bsum(acc):
    lanes = lax.iota(jnp.int32, _L)
    for sh in (8, 4, 2, 1):
        idx = jnp.bitwise_xor(lanes, _full(sh, jnp.int32))
        acc = acc + acc.at[idx].get(mode="promise_in_bounds")
    return acc


def _body(idx_hbm, marg_hbm, table_hbm, out_hbm, idx_v, rows_v, marg_v, out_v, sem):
    at_home = (lax.axis_index("c") == 0) & (lax.axis_index("s") == 0)

    @pl.when(at_home)
    def _():
        pltpu.sync_copy(idx_hbm, idx_v)
        pltpu.sync_copy(marg_hbm, marg_v)
        iv = idx_v[...]
        copies = [
            pltpu.make_async_copy(
                table_hbm.at[pl.ds(iv[k] * _D, _D)], rows_v.at[k], sem
            )
            for k in range(4)
        ]
        for c in copies:
            c.start()
        for c in copies:
            c.wait()

        zero = _full(0.0)
        acc_h, acc_t, acc_c = zero, zero, zero
        hs, rs, ts, cs = [], [], [], []
        for j in range(_NCHUNK):
            sl = pl.ds(j * _L, _L)
            hj = rows_v[0, sl]
            rj = rows_v[1, sl]
            tj = rows_v[2, sl]
            cj = rows_v[3, sl]
            hs.append(hj)
            rs.append(rj)
            ts.append(tj)
            cs.append(cj)
            acc_h = acc_h + hj * hj
            acc_t = acc_t + tj * tj
            acc_c = acc_c + cj * cj

        tiny = _full(1e-30)
        eps = _full(1e-12)
        one = _full(1.0)

        def inv_norm(ssq):
            nrm = ssq * _rsqrt16(jnp.maximum(ssq, tiny))
            return one / jnp.maximum(nrm, eps)

        inv_h = inv_norm(_bsum(acc_h))
        inv_t = inv_norm(_bsum(acc_t))
        inv_c = inv_norm(_bsum(acc_c))

        acc_p, acc_n = zero, zero
        for j in range(_NCHUNK):
            base = rs[j] - ts[j] * inv_t
            d = hs[j] * inv_h + base
            e = cs[j] * inv_c + base
            acc_p = acc_p + d * d
            acc_n = acc_n + e * e

        ssq_p = _bsum(acc_p)
        ssq_n = _bsum(acc_n)
        pos = ssq_p * _rsqrt16(jnp.maximum(ssq_p, tiny))
        neg = ssq_n * _rsqrt16(jnp.maximum(ssq_n, tiny))

        out_v[...] = pos - neg + marg_v[...]
        pltpu.sync_copy(out_v, out_hbm)


_transe_sc = functools.partial(
    pl.kernel,
    mesh=plsc.VectorSubcoreMesh(
        core_axis_name="c", subcore_axis_name="s", num_cores=1
    ),
    out_type=jax.ShapeDtypeStruct((_L,), jnp.float32),
    scratch_types=[
        pltpu.VMEM((_L,), jnp.int32),
        pltpu.VMEM((4, _D), jnp.float32),
        pltpu.VMEM((_L,), jnp.float32),
        pltpu.VMEM((_L,), jnp.float32),
        pltpu.SemaphoreType.DMA,
    ],
)(_body)


def kernel(data, ent_embeds, corrupt_idx, margin):
    idx = jnp.concatenate(
        [data[-1, :], corrupt_idx, jnp.zeros((_L - 4,), jnp.int32)]
    )
    marg = jnp.concatenate([margin, jnp.zeros((_L - 1,), jnp.float32)])
    out = _transe_sc(idx, marg, ent_embeds.reshape(-1))
    return out[:1]


# speedup vs baseline: 1.7215x; 1.7215x over previous
import functools

import jax
import jax.numpy as jnp
from jax import lax
from jax.experimental import pallas as pl
from jax.experimental.pallas import tpu as pltpu
from jax.experimental.pallas import tpu_sc as plsc

_L = 16
_D = 64
_NCHUNK = _D // _L


def _full(v, dtype=jnp.float32):
    return jnp.full((_L,), v, dtype)


def _rsqrt16(v):
    i = lax.bitcast_convert_type(v, jnp.int32)
    i = _full(0x5F3759DF, jnp.int32) - lax.shift_right_arithmetic(
        i, _full(1, jnp.int32)
    )
    y = lax.bitcast_convert_type(i, jnp.float32)
    half_v = _full(0.5) * v
    three_half = _full(1.5)
    for _ in range(4):
        y = y * (three_half - half_v * y * y)
    return y


def _bsum(acc):
    lanes = lax.iota(jnp.int32, _L)
    for sh in (8, 4, 2, 1):
        idx = jnp.bitwise_xor(lanes, _full(sh, jnp.int32))
        acc = acc + acc.at[idx].get(mode="promise_in_bounds")
    return acc


def _body(idx_hbm, marg_hbm, table_hbm, out_hbm, idx_v, rows_v, marg_v, out_v, sem):
    at_home = (lax.axis_index("c") == 0) & (lax.axis_index("s") == 0)

    @pl.when(at_home)
    def _():
        pltpu.sync_copy(idx_hbm, idx_v)
        pltpu.sync_copy(marg_hbm, marg_v)
        iv = idx_v[...]
        copies = [
            pltpu.make_async_copy(table_hbm.at[iv[k]], rows_v.at[k], sem)
            for k in range(4)
        ]
        for c in copies:
            c.start()
        for c in copies:
            c.wait()

        zero = _full(0.0)
        acc_h, acc_t, acc_c = zero, zero, zero
        hs, rs, ts, cs = [], [], [], []
        for j in range(_NCHUNK):
            sl = pl.ds(j * _L, _L)
            hj = rows_v[0, sl]
            rj = rows_v[1, sl]
            tj = rows_v[2, sl]
            cj = rows_v[3, sl]
            hs.append(hj)
            rs.append(rj)
            ts.append(tj)
            cs.append(cj)
            acc_h = acc_h + hj * hj
            acc_t = acc_t + tj * tj
            acc_c = acc_c + cj * cj

        tiny = _full(1e-30)
        eps = _full(1e-12)
        one = _full(1.0)

        def inv_norm(ssq):
            nrm = ssq * _rsqrt16(jnp.maximum(ssq, tiny))
            return one / jnp.maximum(nrm, eps)

        inv_h = inv_norm(_bsum(acc_h))
        inv_t = inv_norm(_bsum(acc_t))
        inv_c = inv_norm(_bsum(acc_c))

        acc_p, acc_n = zero, zero
        for j in range(_NCHUNK):
            base = rs[j] - ts[j] * inv_t
            d = hs[j] * inv_h + base
            e = cs[j] * inv_c + base
            acc_p = acc_p + d * d
            acc_n = acc_n + e * e

        ssq_p = _bsum(acc_p)
        ssq_n = _bsum(acc_n)
        pos = ssq_p * _rsqrt16(jnp.maximum(ssq_p, tiny))
        neg = ssq_n * _rsqrt16(jnp.maximum(ssq_n, tiny))

        out_v[...] = pos - neg + marg_v[...]
        pltpu.sync_copy(out_v, out_hbm)


_transe_sc = functools.partial(
    pl.kernel,
    mesh=plsc.VectorSubcoreMesh(
        core_axis_name="c", subcore_axis_name="s", num_cores=1
    ),
    out_type=jax.ShapeDtypeStruct((_L,), jnp.float32),
    compiler_params=pltpu.CompilerParams(use_tc_tiling_on_sc=True),
    scratch_types=[
        pltpu.VMEM((_L,), jnp.int32),
        pltpu.VMEM((4, _D), jnp.float32),
        pltpu.VMEM((_L,), jnp.float32),
        pltpu.VMEM((_L,), jnp.float32),
        pltpu.SemaphoreType.DMA,
    ],
)(_body)


def kernel(data, ent_embeds, corrupt_idx, margin):
    idx = jnp.concatenate(
        [data[-1, :], corrupt_idx, jnp.zeros((_L - 4,), jnp.int32)]
    )
    marg = jnp.concatenate([margin, jnp.zeros((_L - 1,), jnp.float32)])
    out = _transe_sc(idx, marg, ent_embeds)
    return out[:1]


# speedup vs baseline: 29.8037x; 17.3124x over previous
import functools

import jax
import jax.numpy as jnp
from jax import lax
from jax.experimental import pallas as pl
from jax.experimental.pallas import tpu as pltpu
from jax.experimental.pallas import tpu_sc as plsc

_L = 16
_D = 64
_NCHUNK = _D // _L


def _full(v, dtype=jnp.float32):
    return jnp.full((_L,), v, dtype)


def _rsqrt16(v):
    i = lax.bitcast_convert_type(v, jnp.int32)
    i = _full(0x5F3759DF, jnp.int32) - lax.shift_right_arithmetic(
        i, _full(1, jnp.int32)
    )
    y = lax.bitcast_convert_type(i, jnp.float32)
    half_v = _full(0.5) * v
    three_half = _full(1.5)
    for _ in range(4):
        y = y * (three_half - half_v * y * y)
    return y


def _bsum(acc):
    lanes = lax.iota(jnp.int32, _L)
    for sh in (8, 4, 2, 1):
        idx = jnp.bitwise_xor(lanes, _full(sh, jnp.int32))
        acc = acc + acc.at[idx].get(mode="promise_in_bounds")
    return acc


def _body(
    data_hbm, cidx_hbm, marg_hbm, table_hbm, out_hbm, idx_v, cidx_v, slabs_v, rows_v, marg_v, out_v, sem
):
    at_home = (lax.axis_index("c") == 0) & (lax.axis_index("s") == 0)

    @pl.when(at_home)
    def _():
        pltpu.sync_copy(data_hbm, idx_v.at[pl.ds(0, 3)])
        pltpu.sync_copy(cidx_hbm, cidx_v.at[pl.ds(0, 1)])
        pltpu.sync_copy(marg_hbm, marg_v.at[pl.ds(0, 1)])
        tv = idx_v[...]
        cv = cidx_v[...]
        iv = [tv[0], tv[1], tv[2], cv[0]]
        slab_copies = [
            pltpu.make_async_copy(
                table_hbm.at[:, pl.ds(pl.multiple_of((iv[k] // 128) * 128, 128), 128)],
                slabs_v.at[k],
                sem,
            )
            for k in range(4)
        ]
        for c in slab_copies:
            c.start()
        for c in slab_copies:
            c.wait()
        for k in range(4):
            pltpu.sync_copy(slabs_v.at[k, :, iv[k] % 128], rows_v.at[k])

        zero = _full(0.0)
        acc_h, acc_t, acc_c = zero, zero, zero
        hs, rs, ts, cs = [], [], [], []
        for j in range(_NCHUNK):
            sl = pl.ds(j * _L, _L)
            hj = rows_v[0, sl]
            rj = rows_v[1, sl]
            tj = rows_v[2, sl]
            cj = rows_v[3, sl]
            hs.append(hj)
            rs.append(rj)
            ts.append(tj)
            cs.append(cj)
            acc_h = acc_h + hj * hj
            acc_t = acc_t + tj * tj
            acc_c = acc_c + cj * cj

        tiny = _full(1e-30)
        eps = _full(1e-12)
        one = _full(1.0)

        def inv_norm(ssq):
            nrm = ssq * _rsqrt16(jnp.maximum(ssq, tiny))
            return one / jnp.maximum(nrm, eps)

        inv_h = inv_norm(_bsum(acc_h))
        inv_t = inv_norm(_bsum(acc_t))
        inv_c = inv_norm(_bsum(acc_c))

        acc_p, acc_n = zero, zero
        for j in range(_NCHUNK):
            base = rs[j] - ts[j] * inv_t
            d = hs[j] * inv_h + base
            e = cs[j] * inv_c + base
            acc_p = acc_p + d * d
            acc_n = acc_n + e * e

        ssq_p = _bsum(acc_p)
        ssq_n = _bsum(acc_n)
        pos = ssq_p * _rsqrt16(jnp.maximum(ssq_p, tiny))
        neg = ssq_n * _rsqrt16(jnp.maximum(ssq_n, tiny))

        out_v[...] = pos - neg + marg_v[...]
        pltpu.sync_copy(out_v.at[pl.ds(0, 1)], out_hbm)


_transe_sc = functools.partial(
    pl.kernel,
    mesh=plsc.VectorSubcoreMesh(
        core_axis_name="c", subcore_axis_name="s", num_cores=1
    ),
    out_type=jax.ShapeDtypeStruct((1,), jnp.float32),
    compiler_params=pltpu.CompilerParams(use_tc_tiling_on_sc=True),
    scratch_types=[
        pltpu.VMEM((_L,), jnp.int32),
        pltpu.VMEM((_L,), jnp.int32),
        pltpu.VMEM_SHARED((4, _D, 128), jnp.float32),
        pltpu.VMEM((4, _D), jnp.float32),
        pltpu.VMEM((_L,), jnp.float32),
        pltpu.VMEM((_L,), jnp.float32),
        pltpu.SemaphoreType.DMA,
    ],
)(_body)


def kernel(data, ent_embeds, corrupt_idx, margin):
    return _transe_sc(data[-1], corrupt_idx, margin, ent_embeds.T)


# speedup vs baseline: 30.8559x; 1.0353x over previous
import functools

import jax
import jax.numpy as jnp
from jax import lax
from jax.experimental import pallas as pl
from jax.experimental.pallas import tpu as pltpu
from jax.experimental.pallas import tpu_sc as plsc

_L = 16
_D = 64
_NCHUNK = _D // _L


def _full(v, dtype=jnp.float32):
    return jnp.full((_L,), v, dtype)


def _rsqrt16(v):
    i = lax.bitcast_convert_type(v, jnp.int32)
    i = _full(0x5F3759DF, jnp.int32) - lax.shift_right_arithmetic(
        i, _full(1, jnp.int32)
    )
    y = lax.bitcast_convert_type(i, jnp.float32)
    half_v = _full(0.5) * v
    three_half = _full(1.5)
    for _ in range(4):
        y = y * (three_half - half_v * y * y)
    return y


def _bsum(acc):
    lanes = lax.iota(jnp.int32, _L)
    for sh in (8, 4, 2, 1):
        idx = jnp.bitwise_xor(lanes, _full(sh, jnp.int32))
        acc = acc + acc.at[idx].get(mode="promise_in_bounds")
    return acc


def _body(
    data_hbm, cidx_hbm, marg_hbm, table_hbm, out_hbm, idx_v, cidx_v, slabs_v, rows_v, marg_v, out_v, sem
):
    at_home = (lax.axis_index("c") == 0) & (lax.axis_index("s") == 0)

    @pl.when(at_home)
    def _():
        in_copies = [
            pltpu.make_async_copy(data_hbm, idx_v.at[pl.ds(0, 3)], sem),
            pltpu.make_async_copy(cidx_hbm, cidx_v.at[pl.ds(0, 1)], sem),
            pltpu.make_async_copy(marg_hbm, marg_v.at[pl.ds(0, 1)], sem),
        ]
        for c in in_copies:
            c.start()
        for c in in_copies:
            c.wait()
        tv = idx_v[...]
        cv = cidx_v[...]
        iv = [tv[0], tv[1], tv[2], cv[0]]
        slab_copies = [
            pltpu.make_async_copy(
                table_hbm.at[:, pl.ds(pl.multiple_of((iv[k] // 128) * 128, 128), 128)],
                slabs_v.at[k],
                sem,
            )
            for k in range(4)
        ]
        for c in slab_copies:
            c.start()
        for c in slab_copies:
            c.wait()
        for k in range(4):
            pltpu.sync_copy(slabs_v.at[k, :, iv[k] % 128], rows_v.at[k])

        zero = _full(0.0)
        acc_h, acc_t, acc_c = zero, zero, zero
        hs, rs, ts, cs = [], [], [], []
        for j in range(_NCHUNK):
            sl = pl.ds(j * _L, _L)
            hj = rows_v[0, sl]
            rj = rows_v[1, sl]
            tj = rows_v[2, sl]
            cj = rows_v[3, sl]
            hs.append(hj)
            rs.append(rj)
            ts.append(tj)
            cs.append(cj)
            acc_h = acc_h + hj * hj
            acc_t = acc_t + tj * tj
            acc_c = acc_c + cj * cj

        tiny = _full(1e-30)
        eps = _full(1e-12)
        one = _full(1.0)

        def inv_norm(ssq):
            nrm = ssq * _rsqrt16(jnp.maximum(ssq, tiny))
            return one / jnp.maximum(nrm, eps)

        inv_h = inv_norm(_bsum(acc_h))
        inv_t = inv_norm(_bsum(acc_t))
        inv_c = inv_norm(_bsum(acc_c))

        acc_p, acc_n = zero, zero
        for j in range(_NCHUNK):
            base = rs[j] - ts[j] * inv_t
            d = hs[j] * inv_h + base
            e = cs[j] * inv_c + base
            acc_p = acc_p + d * d
            acc_n = acc_n + e * e

        ssq_p = _bsum(acc_p)
        ssq_n = _bsum(acc_n)
        pos = ssq_p * _rsqrt16(jnp.maximum(ssq_p, tiny))
        neg = ssq_n * _rsqrt16(jnp.maximum(ssq_n, tiny))

        out_v[...] = pos - neg + marg_v[...]
        pltpu.sync_copy(out_v.at[pl.ds(0, 1)], out_hbm)


_transe_sc = functools.partial(
    pl.kernel,
    mesh=plsc.VectorSubcoreMesh(
        core_axis_name="c", subcore_axis_name="s", num_cores=1
    ),
    out_type=jax.ShapeDtypeStruct((1,), jnp.float32),
    compiler_params=pltpu.CompilerParams(use_tc_tiling_on_sc=True),
    scratch_types=[
        pltpu.VMEM((_L,), jnp.int32),
        pltpu.VMEM((_L,), jnp.int32),
        pltpu.VMEM_SHARED((4, _D, 128), jnp.float32),
        pltpu.VMEM((4, _D), jnp.float32),
        pltpu.VMEM((_L,), jnp.float32),
        pltpu.VMEM((_L,), jnp.float32),
        pltpu.SemaphoreType.DMA,
    ],
)(_body)


def kernel(data, ent_embeds, corrupt_idx, margin):
    return _transe_sc(data[-1], corrupt_idx, margin, ent_embeds.T)


# speedup vs baseline: 31.2257x; 1.0120x over previous
import functools

import jax
import jax.numpy as jnp
from jax import lax
from jax.experimental import pallas as pl
from jax.experimental.pallas import tpu as pltpu
from jax.experimental.pallas import tpu_sc as plsc

_L = 16
_D = 64
_NCHUNK = _D // _L


def _full(v, dtype=jnp.float32):
    return jnp.full((_L,), v, dtype)


def _rsqrt16(v):
    i = lax.bitcast_convert_type(v, jnp.int32)
    i = _full(0x5F3759DF, jnp.int32) - lax.shift_right_arithmetic(
        i, _full(1, jnp.int32)
    )
    y = lax.bitcast_convert_type(i, jnp.float32)
    half_v = _full(0.5) * v
    three_half = _full(1.5)
    for _ in range(4):
        y = y * (three_half - half_v * y * y)
    return y


def _bsum(acc):
    lanes = lax.iota(jnp.int32, _L)
    for sh in (8, 4, 2, 1):
        idx = jnp.bitwise_xor(lanes, _full(sh, jnp.int32))
        acc = acc + acc.at[idx].get(mode="promise_in_bounds")
    return acc


def _body(
    data_hbm, cidx_hbm, marg_hbm, table_hbm, out_hbm, idx_v, cidx_v, slabs_v, rows_v, marg_v, out_v, sem
):
    at_home = (lax.axis_index("c") == 0) & (lax.axis_index("s") == 0)

    @pl.when(at_home)
    def _():
        in_copies = [
            pltpu.make_async_copy(data_hbm, idx_v.at[pl.ds(0, 3)], sem),
            pltpu.make_async_copy(cidx_hbm, cidx_v.at[pl.ds(0, 1)], sem),
            pltpu.make_async_copy(marg_hbm, marg_v.at[pl.ds(0, 1)], sem),
        ]
        for c in in_copies:
            c.start()
        for c in in_copies:
            c.wait()
        tv = idx_v[...]
        cv = cidx_v[...]
        iv = [tv[0], tv[1], tv[2], cv[0]]
        slab_copies = [
            pltpu.make_async_copy(
                table_hbm.at[:, pl.ds(pl.multiple_of((iv[k] // 128) * 128, 128), 128)],
                slabs_v.at[k],
                sem,
            )
            for k in range(4)
        ]
        for c in slab_copies:
            c.start()
        for c in slab_copies:
            c.wait()
        col_copies = [
            pltpu.make_async_copy(
                slabs_v.at[k, :, iv[k] % 128], rows_v.at[k], sem
            )
            for k in range(4)
        ]
        for c in col_copies:
            c.start()
        for c in col_copies:
            c.wait()

        zero = _full(0.0)
        acc_h, acc_t, acc_c = zero, zero, zero
        hs, rs, ts, cs = [], [], [], []
        for j in range(_NCHUNK):
            sl = pl.ds(j * _L, _L)
            hj = rows_v[0, sl]
            rj = rows_v[1, sl]
            tj = rows_v[2, sl]
            cj = rows_v[3, sl]
            hs.append(hj)
            rs.append(rj)
            ts.append(tj)
            cs.append(cj)
            acc_h = acc_h + hj * hj
            acc_t = acc_t + tj * tj
            acc_c = acc_c + cj * cj

        tiny = _full(1e-30)
        eps = _full(1e-12)
        one = _full(1.0)

        def inv_norm(ssq):
            nrm = ssq * _rsqrt16(jnp.maximum(ssq, tiny))
            return one / jnp.maximum(nrm, eps)

        inv_h = inv_norm(_bsum(acc_h))
        inv_t = inv_norm(_bsum(acc_t))
        inv_c = inv_norm(_bsum(acc_c))

        acc_p, acc_n = zero, zero
        for j in range(_NCHUNK):
            base = rs[j] - ts[j] * inv_t
            d = hs[j] * inv_h + base
            e = cs[j] * inv_c + base
            acc_p = acc_p + d * d
            acc_n = acc_n + e * e

        ssq_p = _bsum(acc_p)
        ssq_n = _bsum(acc_n)
        pos = ssq_p * _rsqrt16(jnp.maximum(ssq_p, tiny))
        neg = ssq_n * _rsqrt16(jnp.maximum(ssq_n, tiny))

        out_v[...] = pos - neg + marg_v[...]
        pltpu.sync_copy(out_v.at[pl.ds(0, 1)], out_hbm)


_transe_sc = functools.partial(
    pl.kernel,
    mesh=plsc.VectorSubcoreMesh(
        core_axis_name="c", subcore_axis_name="s", num_cores=1
    ),
    out_type=jax.ShapeDtypeStruct((1,), jnp.float32),
    compiler_params=pltpu.CompilerParams(use_tc_tiling_on_sc=True),
    scratch_types=[
        pltpu.VMEM((_L,), jnp.int32),
        pltpu.VMEM((_L,), jnp.int32),
        pltpu.VMEM_SHARED((4, _D, 128), jnp.float32),
        pltpu.VMEM((4, _D), jnp.float32),
        pltpu.VMEM((_L,), jnp.float32),
        pltpu.VMEM((_L,), jnp.float32),
        pltpu.SemaphoreType.DMA,
    ],
)(_body)


def kernel(data, ent_embeds, corrupt_idx, margin):
    return _transe_sc(data[-1], corrupt_idx, margin, ent_embeds.T)
